# baseline (speedup 1.0000x reference)
import functools

import jax
import jax.numpy as jnp
from jax import lax
from jax.experimental import pallas as pl
from jax.experimental.pallas import tpu as pltpu
from jax.experimental.pallas import tpu_sc as plsc

N_NODES = 50000
N_SENSORS = 64
HID = 64
D_MAX = 8.5

NC = 2
NS = 16
NW = NC * NS
CHUNK = 128
NCH = 196
E_PAD = NW * NCH * CHUNK
NSEG = 50048
SEG_SL = NSEG // NS

_sc_mesh = plsc.VectorSubcoreMesh(core_axis_name="c", subcore_axis_name="s")


@functools.partial(
    pl.kernel,
    out_type=jax.ShapeDtypeStruct((NW, NCH, CHUNK), jnp.float32),
    mesh=_sc_mesh,
    scratch_types=[
        pltpu.VMEM((NCH, CHUNK), jnp.float32),
        pltpu.VMEM((NCH, CHUNK), jnp.int32),
        pltpu.VMEM((SEG_SL,), jnp.float32),
        pltpu.VMEM((NSEG,), jnp.float32),
        pltpu.VMEM_SHARED((NSEG,), jnp.float32),
        pltpu.SemaphoreType.DMA,
    ],
    compiler_params=pltpu.CompilerParams(use_tc_tiling_on_sc=False,
                                         needs_layout_passes=False),
)
def _sc_softmax_norm(ex_hbm, dst_hbm, zeros_hbm, out_hbm,
                     val_v, idx_v, zbuf, den_v, acc, sem):
    c = lax.axis_index("c")
    s = lax.axis_index("s")
    wid = s * NC + c
    wid2 = s * NC + (1 - c)
    pltpu.sync_copy(zeros_hbm.at[pl.ds(s * SEG_SL, SEG_SL)], zbuf)
    pltpu.sync_copy(zbuf, acc.at[pl.ds(s * SEG_SL, SEG_SL)])
    pltpu.sync_copy(ex_hbm.at[wid], val_v)
    pltpu.sync_copy(dst_hbm.at[wid], idx_v)
    plsc.subcore_barrier()

    def addbody(g, carry):
        for b in range(7):
            j = g * 7 + b
            pltpu.async_copy(val_v.at[j], acc.at[idx_v.at[j]], sem, add=True)
        for b in range(7):
            j = g * 7 + b
            pltpu.make_async_copy(val_v.at[j], acc.at[idx_v.at[j]], sem).wait()
        return carry

    lax.fori_loop(0, NCH // 7, addbody, 0)
    pltpu.sync_copy(ex_hbm.at[wid2], val_v)
    pltpu.sync_copy(dst_hbm.at[wid2], idx_v)
    lax.fori_loop(0, NCH // 7, addbody, 0)
    plsc.subcore_barrier()
    pltpu.sync_copy(acc, den_v)
    pltpu.sync_copy(ex_hbm.at[wid], val_v)
    pltpu.sync_copy(dst_hbm.at[wid], idx_v)

    def normbody(j, carry):
        for k in range(CHUNK // 16):
            sl = pl.ds(k * 16, 16)
            idx = idx_v[j, sl]
            d = plsc.load_gather(den_v, [idx])
            val_v[j, sl] = val_v[j, sl] / (d + 1e-16)
        return carry

    lax.fori_loop(0, NCH, normbody, 0)
    pltpu.sync_copy(val_v, out_hbm.at[wid])


EPT = NCH * CHUNK
NBUF = 4


@functools.partial(
    pl.kernel,
    out_type=jax.ShapeDtypeStruct((NW, EPT, HID), jnp.float32),
    mesh=_sc_mesh,
    scratch_types=[
        pltpu.VMEM((NCH, CHUNK), jnp.int32),
        [pltpu.VMEM((CHUNK, HID), jnp.float32) for _ in range(NBUF)],
        pltpu.SemaphoreType.DMA,
        pltpu.SemaphoreType.DMA,
    ],
    compiler_params=pltpu.CompilerParams(use_tc_tiling_on_sc=False),
)
def _sc_gather(table_hbm, idx_hbm, out_hbm, idx_v, bufs, gsem, ssem):
    c = lax.axis_index("c")
    s = lax.axis_index("s")
    wid = s * NC + c
    pltpu.sync_copy(idx_hbm.at[wid], idx_v)

    def body(g, carry):
        for b in range(NBUF):
            j = g * NBUF + b
            pltpu.async_copy(table_hbm.at[idx_v.at[j]], bufs[b], gsem)
        for b in range(NBUF):
            j = g * NBUF + b
            pltpu.make_async_copy(table_hbm.at[idx_v.at[j]], bufs[b], gsem).wait()
            pltpu.async_copy(bufs[b], out_hbm.at[wid, pl.ds(j * CHUNK, CHUNK)], ssem)
        for b in range(NBUF):
            j = g * NBUF + b
            pltpu.make_async_copy(bufs[b], out_hbm.at[wid, pl.ds(j * CHUNK, CHUNK)], ssem).wait()
        return carry

    lax.fori_loop(0, NCH // NBUF, body, 0)


def _gather_rows_sc(table, idx_3d):
    return _sc_gather(table, idx_3d).reshape(E_PAD, HID)


HHID = HID // 4
FL_SL = SEG_SL // 2


@functools.partial(
    pl.kernel,
    out_type=jax.ShapeDtypeStruct((NC, 2, NSEG, HHID), jnp.float32),
    mesh=_sc_mesh,
    scratch_types=[
        pltpu.VMEM((NCH, CHUNK), jnp.int32),
        [pltpu.VMEM((CHUNK, HHID), jnp.float32) for _ in range(NBUF)],
        pltpu.VMEM((FL_SL, HHID), jnp.float32),
        pltpu.VMEM_SHARED((NSEG, HHID), jnp.float32),
        pltpu.SemaphoreType.DMA,
        pltpu.SemaphoreType.DMA,
    ],
    compiler_params=pltpu.CompilerParams(use_tc_tiling_on_sc=False),
)
def _sc_segsum_rows(za_hbm, dst_hbm, zeros_hbm, out_hbm,
                    idx_v, bufs, zbuf, acc, rsem, asem):
    c = lax.axis_index("c")
    s = lax.axis_index("s")

    def make_slice_body(base, col0):
        def body(g, carry):
            for b in range(NBUF):
                j = g * NBUF + b
                src = za_hbm.at[pl.ds(base + j * CHUNK, CHUNK),
                                pl.ds(col0, HHID)]
                pltpu.async_copy(src, bufs[b], rsem)
            for b in range(NBUF):
                j = g * NBUF + b
                src = za_hbm.at[pl.ds(base + j * CHUNK, CHUNK),
                                pl.ds(col0, HHID)]
                pltpu.make_async_copy(src, bufs[b], rsem).wait()
                pltpu.async_copy(bufs[b], acc.at[idx_v.at[j]], asem, add=True)
            for b in range(NBUF):
                j = g * NBUF + b
                pltpu.make_async_copy(
                    bufs[b], acc.at[idx_v.at[j]], asem).wait()
            return carry

        return body

    for q in range(2):
        col0 = pl.multiple_of((c * 2 + q) * HHID, 8)
        pltpu.sync_copy(zeros_hbm.at[pl.ds(0, FL_SL)], zbuf)
        for half in range(2):
            pltpu.sync_copy(
                zbuf, acc.at[pl.ds(s * SEG_SL + half * FL_SL, FL_SL)])
        plsc.subcore_barrier()
        for wid in (s * NC + c, s * NC + (1 - c)):
            pltpu.sync_copy(dst_hbm.at[wid], idx_v)
            lax.fori_loop(0, NCH // NBUF, make_slice_body(wid * EPT, col0), 0)
        plsc.subcore_barrier()
        for half in range(2):
            row0 = s * SEG_SL + half * FL_SL
            pltpu.sync_copy(acc.at[pl.ds(row0, FL_SL)], zbuf)
            pltpu.sync_copy(zbuf, out_hbm.at[c, q, pl.ds(row0, FL_SL)])


def _segsum_rows_sc(za, dst_3d, zeros_half):
    res = _sc_segsum_rows(za, dst_3d, zeros_half)
    return jnp.concatenate(
        [res[0, 0], res[0, 1], res[1, 0], res[1, 1]], axis=1)[:N_NODES]


def _softmax_norm_sc(ex, dst_3d, zeros_seg):
    ex_3d = ex.reshape(NW, NCH, CHUNK)
    return _sc_softmax_norm(ex_3d, dst_3d, zeros_seg).reshape(E_PAD)


def _linear(x, W, b):
    return x @ W.T + b


def _layernorm(x, g, b, eps=1e-5):
    m = jnp.mean(x, axis=-1, keepdims=True)
    v = jnp.mean((x - m) ** 2, axis=-1, keepdims=True)
    return (x - m) / jnp.sqrt(v + eps) * g + b


def _leaky(x, s):
    return jnp.where(x >= 0, x, s * x)


def _graph_norm(x, w, b, ms, eps=1e-5):
    mean = jnp.mean(x, axis=0, keepdims=True)
    out = x - ms * mean
    var = jnp.mean(out ** 2, axis=0, keepdims=True)
    return w * out / jnp.sqrt(var + eps) + b


EBLK = 8192
EGRID = E_PAD // EBLK


def _edge_alpha_body(g_ref, ea_ref, wet_ref, be_ref, attn_ref, msk_ref, ex_ref):
    e = jnp.dot(ea_ref[...], wet_ref[...], preferred_element_type=jnp.float32)
    z = g_ref[...] * jax.nn.sigmoid(e + be_ref[...])
    alpha = jnp.sum(z * attn_ref[...], axis=-1)
    alpha = jnp.where(alpha >= 0, alpha, 0.1 * alpha)
    ex_ref[...] = jnp.exp(alpha) * msk_ref[...]


def _edge_alpha_tc(G, ea_pad, We, be, attn, padmask):
    return pl.pallas_call(
        _edge_alpha_body,
        grid=(EGRID,),
        in_specs=[
            pl.BlockSpec((EBLK, HID), lambda i: (i, 0)),
            pl.BlockSpec((EBLK, 3), lambda i: (i, 0)),
            pl.BlockSpec((3, HID), lambda i: (0, 0)),
            pl.BlockSpec((1, HID), lambda i: (0, 0)),
            pl.BlockSpec((1, HID), lambda i: (0, 0)),
            pl.BlockSpec((EBLK,), lambda i: (i,)),
        ],
        out_specs=pl.BlockSpec((EBLK,), lambda i: (i,)),
        out_shape=jax.ShapeDtypeStruct((E_PAD,), jnp.float32),
    )(G, ea_pad, We.T, be.reshape(1, HID), attn.reshape(1, HID), padmask)


def _edge_za_body(g_ref, ea_ref, wet_ref, be_ref, a_ref, za_ref):
    e = jnp.dot(ea_ref[...], wet_ref[...], preferred_element_type=jnp.float32)
    z = g_ref[...] * jax.nn.sigmoid(e + be_ref[...])
    za_ref[...] = z * a_ref[...][:, None]


def _edge_za_tc(G, ea_pad, We, be, a):
    return pl.pallas_call(
        _edge_za_body,
        grid=(EGRID,),
        in_specs=[
            pl.BlockSpec((EBLK, HID), lambda i: (i, 0)),
            pl.BlockSpec((EBLK, 3), lambda i: (i, 0)),
            pl.BlockSpec((3, HID), lambda i: (0, 0)),
            pl.BlockSpec((1, HID), lambda i: (0, 0)),
            pl.BlockSpec((EBLK,), lambda i: (i,)),
        ],
        out_specs=pl.BlockSpec((EBLK, HID), lambda i: (i, 0)),
        out_shape=jax.ShapeDtypeStruct((E_PAD, HID), jnp.float32),
    )(G, ea_pad, We.T, be.reshape(1, HID), a)


def _klat_body(gd_ref, z_ref, scale_ref, g_ref, b_ref, out_ref):
    gd = gd_ref[...]
    scale = scale_ref[0]
    k = jnp.exp(-gd / (scale + 1e-06))
    k = jnp.where(gd > D_MAX, 0.0, k)
    m = jnp.dot(k, z_ref[...], preferred_element_type=jnp.float32)
    mu = jnp.mean(m, axis=-1, keepdims=True)
    var = jnp.mean((m - mu) ** 2, axis=-1, keepdims=True)
    out_ref[...] = (m - mu) / jnp.sqrt(var + 1e-5) * g_ref[...] + b_ref[...]


def _k_latent(gd, z_sensors, scale, g, b):
    blk = 1000
    grid = (N_NODES // blk,)
    return pl.pallas_call(
        _klat_body,
        grid=grid,
        in_specs=[
            pl.BlockSpec((blk, N_SENSORS), lambda i: (i, 0)),
            pl.BlockSpec((N_SENSORS, HID), lambda i: (0, 0)),
            pl.BlockSpec(memory_space=pltpu.SMEM),
            pl.BlockSpec((1, HID), lambda i: (0, 0)),
            pl.BlockSpec((1, HID), lambda i: (0, 0)),
        ],
        out_specs=pl.BlockSpec((blk, HID), lambda i: (i, 0)),
        out_shape=jax.ShapeDtypeStruct((N_NODES, HID), jnp.float32),
    )(gd, z_sensors, scale.reshape(1), g.reshape(1, HID), b.reshape(1, HID))


def _gat_layer(x, edge_attr_pad, p, pre, src_3d, dst_3d,
               padmask, zeros_seg, zeros_half):
    h = _linear(x, p[pre + '_Wn'], p[pre + '_bn'])
    G = _gather_rows_sc(h, src_3d)
    We, be, attn = p[pre + '_We'], p[pre + '_be'], p[pre + '_attn']
    ex = _edge_alpha_tc(G, edge_attr_pad, We, be, attn, padmask)
    a = _softmax_norm_sc(ex, dst_3d, zeros_seg)
    za = _edge_za_tc(G, edge_attr_pad, We, be, a)
    return _segsum_rows_sc(za, dst_3d, zeros_half)


def _sensor_encoder(input_seq, p):
    S, T, _ = input_seq.shape
    h = jnp.zeros((S, HID), dtype=input_seq.dtype)
    h_new = h
    for t in range(T):
        x_t = _linear(input_seq[:, t, 0:1], p['enc_in_W'], p['enc_in_b'])
        delta = input_seq[:, t, 1]
        dh = (-h + jnp.tanh(x_t + _linear(h, p['enc_Wh_W'], p['enc_Wh_b']))) / p['enc_tau']
        h_new = h + delta[:, None] * dh
    return jnp.tanh(_linear(h_new, p['enc_out_W'], p['enc_out_b']))


def _w_xyz_mlp(x, p):
    h = jnp.tanh(_layernorm(_linear(x, p['wxyz_W1'], p['wxyz_b1']), p['wxyz_g1'], p['wxyz_e1']))
    h = jnp.tanh(_linear(h, p['wxyz_W2'], p['wxyz_b2']))
    return _linear(h, p['wxyz_W3'], p['wxyz_b3'])


def _w_type_mlp(x, p):
    h = jnp.tanh(_layernorm(_linear(x, p['wtype_W1'], p['wtype_b1']), p['wtype_g1'], p['wtype_e1']))
    h = jnp.tanh(_linear(h, p['wtype_W2'], p['wtype_b2']))
    return _linear(h, p['wtype_W3'], p['wtype_b3'])


def kernel(x, x_branch, geodesic_dist, ini_GNN_temp, edge_index, edge_attr, params):
    p = params
    temp_seq = x_branch[:, 6:]
    T = temp_seq.shape[1]
    delta_t = jnp.asarray([2.0] * (T - 1) + [120.0], dtype=jnp.float32)
    delta_t = delta_t / jnp.max(delta_t)
    delta_t = jnp.broadcast_to(delta_t[None, :], temp_seq.shape)
    input_seq = jnp.stack([temp_seq, delta_t], axis=-1)
    z_sensors = _sensor_encoder(input_seq, p)
    x_latent = _k_latent(geodesic_dist, z_sensors, p['kernel_scale'],
                         p['ln_lat_g'], p['ln_lat_b'])
    x_keep = x[:, jnp.asarray([0, 1, 2, 4, 5, 6])]
    xyz = x_keep[:, 0:3]
    x_xyz_w = _w_xyz_mlp(xyz, p) * xyz
    x_type = x_keep[:, 3:6]
    x_type_w = _w_type_mlp(x_type, p) * x_type
    x_ini = ini_GNN_temp * p['weights']
    x_all = jnp.concatenate([x_xyz_w, x_type_w, x_ini, x_latent], axis=1)
    n = x.shape[0]
    n_pad = E_PAD - edge_index.shape[1]
    pad = jnp.zeros((n_pad,), jnp.int32)
    src_3d = jnp.concatenate([edge_index[0], pad]).reshape(NW, NCH, CHUNK)
    dst_3d = jnp.concatenate([edge_index[1], pad]).reshape(NW, NCH, CHUNK)
    edge_attr_pad = jnp.concatenate(
        [edge_attr, jnp.zeros((n_pad, 3), jnp.float32)])
    padmask = jnp.concatenate([jnp.ones((edge_index.shape[1],), jnp.float32),
                               jnp.zeros((n_pad,), jnp.float32)])
    zeros_seg = jnp.zeros((NSEG,), jnp.float32)
    zeros_half = jnp.zeros((FL_SL, HHID), jnp.float32)
    gl = functools.partial(_gat_layer, edge_attr_pad=edge_attr_pad, p=p,
                           src_3d=src_3d, dst_3d=dst_3d, padmask=padmask,
                           zeros_seg=zeros_seg, zeros_half=zeros_half)
    h1 = _leaky(_graph_norm(gl(x_all, pre='g1'), p['n1_w'], p['n1_b'], p['n1_ms']), 0.01)
    h2 = _leaky(_graph_norm(gl(h1, pre='g2'), p['n2_w'], p['n2_b'], p['n2_ms']), 0.01) + h1
    h3 = _leaky(_graph_norm(gl(h2, pre='g3'), p['n3_w'], p['n3_b'], p['n3_ms']), 0.01) + h2
    h4 = _leaky(_graph_norm(gl(h3, pre='g4'), p['n4_w'], p['n4_b'], p['n4_ms']), 0.01) + h3
    h5 = _leaky(_graph_norm(gl(h4, pre='g5'), p['n5_w'], p['n5_b'], p['n5_ms']), 0.01) + h4
    o = _leaky(_layernorm(_linear(h5, p['out_W1'], p['out_b1']), p['out_g1'], p['out_e1']), 0.01)
    o = jnp.tanh(_linear(o, p['out_W2'], p['out_b2']))
    return o[:, 0]


# speedup vs baseline: 1.4594x; 1.4594x over previous
import functools

import jax
import jax.numpy as jnp
from jax import lax
from jax.experimental import pallas as pl
from jax.experimental.pallas import tpu as pltpu
from jax.experimental.pallas import tpu_sc as plsc

N_NODES = 50000
N_SENSORS = 64
HID = 64
D_MAX = 8.5

NC = 2
NS = 16
NW = NC * NS
CHUNK = 128
NCH = 196
E_PAD = NW * NCH * CHUNK
NSEG = 50048
SEG_SL = NSEG // NS

_sc_mesh = plsc.VectorSubcoreMesh(core_axis_name="c", subcore_axis_name="s")


@functools.partial(
    pl.kernel,
    out_type=jax.ShapeDtypeStruct((NW, NCH, CHUNK), jnp.float32),
    mesh=_sc_mesh,
    scratch_types=[
        pltpu.VMEM((NCH, CHUNK), jnp.float32),
        pltpu.VMEM((NCH, CHUNK), jnp.int32),
        pltpu.VMEM((SEG_SL,), jnp.float32),
        pltpu.VMEM((NSEG,), jnp.float32),
        pltpu.VMEM_SHARED((NSEG,), jnp.float32),
        pltpu.SemaphoreType.DMA,
    ],
    compiler_params=pltpu.CompilerParams(use_tc_tiling_on_sc=False,
                                         needs_layout_passes=False),
)
def _sc_softmax_norm(ex_hbm, dst_hbm, zeros_hbm, out_hbm,
                     val_v, idx_v, zbuf, den_v, acc, sem):
    c = lax.axis_index("c")
    s = lax.axis_index("s")
    wid = s * NC + c
    wid2 = s * NC + (1 - c)
    pltpu.sync_copy(zeros_hbm.at[pl.ds(s * SEG_SL, SEG_SL)], zbuf)
    pltpu.sync_copy(zbuf, acc.at[pl.ds(s * SEG_SL, SEG_SL)])
    pltpu.sync_copy(ex_hbm.at[wid], val_v)
    pltpu.sync_copy(dst_hbm.at[wid], idx_v)
    plsc.subcore_barrier()

    def addbody(g, carry):
        for b in range(7):
            j = g * 7 + b
            pltpu.async_copy(val_v.at[j], acc.at[idx_v.at[j]], sem, add=True)
        for b in range(7):
            j = g * 7 + b
            pltpu.make_async_copy(val_v.at[j], acc.at[idx_v.at[j]], sem).wait()
        return carry

    lax.fori_loop(0, NCH // 7, addbody, 0)
    pltpu.sync_copy(ex_hbm.at[wid2], val_v)
    pltpu.sync_copy(dst_hbm.at[wid2], idx_v)
    lax.fori_loop(0, NCH // 7, addbody, 0)
    plsc.subcore_barrier()
    pltpu.sync_copy(acc, den_v)
    pltpu.sync_copy(ex_hbm.at[wid], val_v)
    pltpu.sync_copy(dst_hbm.at[wid], idx_v)

    def normbody(j, carry):
        for k in range(CHUNK // 16):
            sl = pl.ds(k * 16, 16)
            idx = idx_v[j, sl]
            d = plsc.load_gather(den_v, [idx])
            val_v[j, sl] = val_v[j, sl] / (d + 1e-16)
        return carry

    lax.fori_loop(0, NCH, normbody, 0)
    pltpu.sync_copy(val_v, out_hbm.at[wid])


EPT = NCH * CHUNK
NBUF = 7


@functools.partial(
    pl.kernel,
    out_type=jax.ShapeDtypeStruct((NW, EPT, HID), jnp.float32),
    mesh=_sc_mesh,
    scratch_types=[
        pltpu.VMEM((NCH, CHUNK), jnp.int32),
        [pltpu.VMEM((CHUNK, HID), jnp.float32) for _ in range(NBUF)],
        pltpu.SemaphoreType.DMA,
        pltpu.SemaphoreType.DMA,
    ],
    compiler_params=pltpu.CompilerParams(use_tc_tiling_on_sc=False),
)
def _sc_gather(table_hbm, idx_hbm, out_hbm, idx_v, bufs, gsem, ssem):
    c = lax.axis_index("c")
    s = lax.axis_index("s")
    wid = s * NC + c
    pltpu.sync_copy(idx_hbm.at[wid], idx_v)

    def body(g, carry):
        for b in range(NBUF):
            j = g * NBUF + b
            pltpu.async_copy(table_hbm.at[idx_v.at[j]], bufs[b], gsem)
        for b in range(NBUF):
            j = g * NBUF + b
            pltpu.make_async_copy(table_hbm.at[idx_v.at[j]], bufs[b], gsem).wait()
            pltpu.async_copy(bufs[b], out_hbm.at[wid, pl.ds(j * CHUNK, CHUNK)], ssem)
        for b in range(NBUF):
            j = g * NBUF + b
            pltpu.make_async_copy(bufs[b], out_hbm.at[wid, pl.ds(j * CHUNK, CHUNK)], ssem).wait()
        return carry

    lax.fori_loop(0, NCH // NBUF, body, 0)


def _gather_rows_sc(table, idx_3d):
    return _sc_gather(table, idx_3d).reshape(E_PAD, HID)


HHID = HID // 4
FL_SL = SEG_SL // 2


@functools.partial(
    pl.kernel,
    out_type=jax.ShapeDtypeStruct((NC, 2, NSEG, HHID), jnp.float32),
    mesh=_sc_mesh,
    scratch_types=[
        pltpu.VMEM((NCH, CHUNK), jnp.int32),
        [pltpu.VMEM((CHUNK, HHID), jnp.float32) for _ in range(NBUF)],
        pltpu.VMEM((FL_SL, HHID), jnp.float32),
        pltpu.VMEM_SHARED((NSEG, HHID), jnp.float32),
        pltpu.SemaphoreType.DMA,
        pltpu.SemaphoreType.DMA,
    ],
    compiler_params=pltpu.CompilerParams(use_tc_tiling_on_sc=False),
)
def _sc_segsum_rows(za_hbm, dst_hbm, zeros_hbm, out_hbm,
                    idx_v, bufs, zbuf, acc, rsem, asem):
    c = lax.axis_index("c")
    s = lax.axis_index("s")

    def make_slice_body(base, col0):
        def body(g, carry):
            for b in range(NBUF):
                j = g * NBUF + b
                src = za_hbm.at[pl.ds(base + j * CHUNK, CHUNK),
                                pl.ds(col0, HHID)]
                pltpu.async_copy(src, bufs[b], rsem)
            for b in range(NBUF):
                j = g * NBUF + b
                src = za_hbm.at[pl.ds(base + j * CHUNK, CHUNK),
                                pl.ds(col0, HHID)]
                pltpu.make_async_copy(src, bufs[b], rsem).wait()
                pltpu.async_copy(bufs[b], acc.at[idx_v.at[j]], asem, add=True)
            for b in range(NBUF):
                j = g * NBUF + b
                pltpu.make_async_copy(
                    bufs[b], acc.at[idx_v.at[j]], asem).wait()
            return carry

        return body

    for q in range(2):
        col0 = pl.multiple_of((c * 2 + q) * HHID, 8)
        pltpu.sync_copy(zeros_hbm.at[pl.ds(0, FL_SL)], zbuf)
        for half in range(2):
            pltpu.sync_copy(
                zbuf, acc.at[pl.ds(s * SEG_SL + half * FL_SL, FL_SL)])
        plsc.subcore_barrier()
        for wid in (s * NC + c, s * NC + (1 - c)):
            pltpu.sync_copy(dst_hbm.at[wid], idx_v)
            lax.fori_loop(0, NCH // NBUF, make_slice_body(wid * EPT, col0), 0)
        plsc.subcore_barrier()
        for half in range(2):
            row0 = s * SEG_SL + half * FL_SL
            pltpu.sync_copy(acc.at[pl.ds(row0, FL_SL)], zbuf)
            pltpu.sync_copy(zbuf, out_hbm.at[c, q, pl.ds(row0, FL_SL)])


def _segsum_rows_sc(za, dst_3d, zeros_half):
    res = _sc_segsum_rows(za, dst_3d, zeros_half)
    return jnp.concatenate(
        [res[0, 0], res[0, 1], res[1, 0], res[1, 1]], axis=1)[:N_NODES]


def _softmax_norm_sc(ex, dst_3d, zeros_seg):
    ex_3d = ex.reshape(NW, NCH, CHUNK)
    return _sc_softmax_norm(ex_3d, dst_3d, zeros_seg).reshape(E_PAD)


def _linear(x, W, b):
    return x @ W.T + b


def _layernorm(x, g, b, eps=1e-5):
    m = jnp.mean(x, axis=-1, keepdims=True)
    v = jnp.mean((x - m) ** 2, axis=-1, keepdims=True)
    return (x - m) / jnp.sqrt(v + eps) * g + b


def _leaky(x, s):
    return jnp.where(x >= 0, x, s * x)


def _graph_norm(x, w, b, ms, eps=1e-5):
    mean = jnp.mean(x, axis=0, keepdims=True)
    out = x - ms * mean
    var = jnp.mean(out ** 2, axis=0, keepdims=True)
    return w * out / jnp.sqrt(var + eps) + b


def _klat_body(gd_ref, z_ref, scale_ref, g_ref, b_ref, out_ref):
    gd = gd_ref[...]
    scale = scale_ref[0]
    k = jnp.exp(-gd / (scale + 1e-06))
    k = jnp.where(gd > D_MAX, 0.0, k)
    m = jnp.dot(k, z_ref[...], preferred_element_type=jnp.float32)
    mu = jnp.mean(m, axis=-1, keepdims=True)
    var = jnp.mean((m - mu) ** 2, axis=-1, keepdims=True)
    out_ref[...] = (m - mu) / jnp.sqrt(var + 1e-5) * g_ref[...] + b_ref[...]


def _k_latent(gd, z_sensors, scale, g, b):
    blk = 1000
    grid = (N_NODES // blk,)
    return pl.pallas_call(
        _klat_body,
        grid=grid,
        in_specs=[
            pl.BlockSpec((blk, N_SENSORS), lambda i: (i, 0)),
            pl.BlockSpec((N_SENSORS, HID), lambda i: (0, 0)),
            pl.BlockSpec(memory_space=pltpu.SMEM),
            pl.BlockSpec((1, HID), lambda i: (0, 0)),
            pl.BlockSpec((1, HID), lambda i: (0, 0)),
        ],
        out_specs=pl.BlockSpec((blk, HID), lambda i: (i, 0)),
        out_shape=jax.ShapeDtypeStruct((N_NODES, HID), jnp.float32),
    )(gd, z_sensors, scale.reshape(1), g.reshape(1, HID), b.reshape(1, HID))


def _gat_layer(x, edge_attr_pad, p, pre, src_3d, dst_3d,
               padmask, zeros_seg, zeros_half):
    h = _linear(x, p[pre + '_Wn'], p[pre + '_bn'])
    e = _linear(edge_attr_pad, p[pre + '_We'], p[pre + '_be'])
    z = _gather_rows_sc(h, src_3d) * jax.nn.sigmoid(e)
    alpha = jnp.sum(z * p[pre + '_attn'], axis=-1)
    alpha = _leaky(alpha, 0.1)
    ex = jnp.exp(alpha) * padmask
    a = _softmax_norm_sc(ex, dst_3d, zeros_seg)
    return _segsum_rows_sc(z * a[:, None], dst_3d, zeros_half)


def _sensor_encoder(input_seq, p):
    S, T, _ = input_seq.shape
    h = jnp.zeros((S, HID), dtype=input_seq.dtype)
    h_new = h
    for t in range(T):
        x_t = _linear(input_seq[:, t, 0:1], p['enc_in_W'], p['enc_in_b'])
        delta = input_seq[:, t, 1]
        dh = (-h + jnp.tanh(x_t + _linear(h, p['enc_Wh_W'], p['enc_Wh_b']))) / p['enc_tau']
        h_new = h + delta[:, None] * dh
    return jnp.tanh(_linear(h_new, p['enc_out_W'], p['enc_out_b']))


def _w_xyz_mlp(x, p):
    h = jnp.tanh(_layernorm(_linear(x, p['wxyz_W1'], p['wxyz_b1']), p['wxyz_g1'], p['wxyz_e1']))
    h = jnp.tanh(_linear(h, p['wxyz_W2'], p['wxyz_b2']))
    return _linear(h, p['wxyz_W3'], p['wxyz_b3'])


def _w_type_mlp(x, p):
    h = jnp.tanh(_layernorm(_linear(x, p['wtype_W1'], p['wtype_b1']), p['wtype_g1'], p['wtype_e1']))
    h = jnp.tanh(_linear(h, p['wtype_W2'], p['wtype_b2']))
    return _linear(h, p['wtype_W3'], p['wtype_b3'])


def kernel(x, x_branch, geodesic_dist, ini_GNN_temp, edge_index, edge_attr, params):
    p = params
    temp_seq = x_branch[:, 6:]
    T = temp_seq.shape[1]
    delta_t = jnp.asarray([2.0] * (T - 1) + [120.0], dtype=jnp.float32)
    delta_t = delta_t / jnp.max(delta_t)
    delta_t = jnp.broadcast_to(delta_t[None, :], temp_seq.shape)
    input_seq = jnp.stack([temp_seq, delta_t], axis=-1)
    z_sensors = _sensor_encoder(input_seq, p)
    x_latent = _k_latent(geodesic_dist, z_sensors, p['kernel_scale'],
                         p['ln_lat_g'], p['ln_lat_b'])
    x_keep = x[:, jnp.asarray([0, 1, 2, 4, 5, 6])]
    xyz = x_keep[:, 0:3]
    x_xyz_w = _w_xyz_mlp(xyz, p) * xyz
    x_type = x_keep[:, 3:6]
    x_type_w = _w_type_mlp(x_type, p) * x_type
    x_ini = ini_GNN_temp * p['weights']
    x_all = jnp.concatenate([x_xyz_w, x_type_w, x_ini, x_latent], axis=1)
    n = x.shape[0]
    n_pad = E_PAD - edge_index.shape[1]
    pad = jnp.zeros((n_pad,), jnp.int32)
    src_3d = jnp.concatenate([edge_index[0], pad]).reshape(NW, NCH, CHUNK)
    dst_3d = jnp.concatenate([edge_index[1], pad]).reshape(NW, NCH, CHUNK)
    edge_attr_pad = jnp.concatenate(
        [edge_attr, jnp.zeros((n_pad, 3), jnp.float32)])
    padmask = jnp.concatenate([jnp.ones((edge_index.shape[1],), jnp.float32),
                               jnp.zeros((n_pad,), jnp.float32)])
    zeros_seg = jnp.zeros((NSEG,), jnp.float32)
    zeros_half = jnp.zeros((FL_SL, HHID), jnp.float32)
    gl = functools.partial(_gat_layer, edge_attr_pad=edge_attr_pad, p=p,
                           src_3d=src_3d, dst_3d=dst_3d, padmask=padmask,
                           zeros_seg=zeros_seg, zeros_half=zeros_half)
    h1 = _leaky(_graph_norm(gl(x_all, pre='g1'), p['n1_w'], p['n1_b'], p['n1_ms']), 0.01)
    h2 = _leaky(_graph_norm(gl(h1, pre='g2'), p['n2_w'], p['n2_b'], p['n2_ms']), 0.01) + h1
    h3 = _leaky(_graph_norm(gl(h2, pre='g3'), p['n3_w'], p['n3_b'], p['n3_ms']), 0.01) + h2
    h4 = _leaky(_graph_norm(gl(h3, pre='g4'), p['n4_w'], p['n4_b'], p['n4_ms']), 0.01) + h3
    h5 = _leaky(_graph_norm(gl(h4, pre='g5'), p['n5_w'], p['n5_b'], p['n5_ms']), 0.01) + h4
    o = _leaky(_layernorm(_linear(h5, p['out_W1'], p['out_b1']), p['out_g1'], p['out_e1']), 0.01)
    o = jnp.tanh(_linear(o, p['out_W2'], p['out_b2']))
    return o[:, 0]


# speedup vs baseline: 1.4900x; 1.0209x over previous
import functools

import jax
import jax.numpy as jnp
from jax import lax
from jax.experimental import pallas as pl
from jax.experimental.pallas import tpu as pltpu
from jax.experimental.pallas import tpu_sc as plsc

N_NODES = 50000
N_SENSORS = 64
HID = 64
D_MAX = 8.5

NC = 2
NS = 16
NW = NC * NS
CHUNK = 128
NCH = 196
E_PAD = NW * NCH * CHUNK
NSEG = 50048
SEG_SL = NSEG // NS

_sc_mesh = plsc.VectorSubcoreMesh(core_axis_name="c", subcore_axis_name="s")


@functools.partial(
    pl.kernel,
    out_type=jax.ShapeDtypeStruct((NW, NCH, CHUNK), jnp.float32),
    mesh=_sc_mesh,
    scratch_types=[
        pltpu.VMEM((NCH, CHUNK), jnp.float32),
        pltpu.VMEM((NCH, CHUNK), jnp.int32),
        pltpu.VMEM((SEG_SL,), jnp.float32),
        pltpu.VMEM((NSEG,), jnp.float32),
        pltpu.VMEM_SHARED((NSEG,), jnp.float32),
        pltpu.SemaphoreType.DMA,
    ],
    compiler_params=pltpu.CompilerParams(use_tc_tiling_on_sc=False,
                                         needs_layout_passes=False),
)
def _sc_softmax_norm(ex_hbm, dst_hbm, zeros_hbm, out_hbm,
                     val_v, idx_v, zbuf, den_v, acc, sem):
    c = lax.axis_index("c")
    s = lax.axis_index("s")
    wid = s * NC + c
    wid2 = s * NC + (1 - c)
    pltpu.sync_copy(zeros_hbm.at[pl.ds(s * SEG_SL, SEG_SL)], zbuf)
    pltpu.sync_copy(zbuf, acc.at[pl.ds(s * SEG_SL, SEG_SL)])
    pltpu.sync_copy(ex_hbm.at[wid], val_v)
    pltpu.sync_copy(dst_hbm.at[wid], idx_v)
    plsc.subcore_barrier()

    def addbody(g, carry):
        for b in range(14):
            j = g * 14 + b
            pltpu.async_copy(val_v.at[j], acc.at[idx_v.at[j]], sem, add=True)
        for b in range(14):
            j = g * 14 + b
            pltpu.make_async_copy(val_v.at[j], acc.at[idx_v.at[j]], sem).wait()
        return carry

    lax.fori_loop(0, NCH // 14, addbody, 0)
    pltpu.sync_copy(ex_hbm.at[wid2], val_v)
    pltpu.sync_copy(dst_hbm.at[wid2], idx_v)
    lax.fori_loop(0, NCH // 14, addbody, 0)
    plsc.subcore_barrier()
    pltpu.sync_copy(acc, den_v)
    pltpu.sync_copy(ex_hbm.at[wid], val_v)
    pltpu.sync_copy(dst_hbm.at[wid], idx_v)

    def normbody(j, carry):
        for k in range(CHUNK // 16):
            sl = pl.ds(k * 16, 16)
            idx = idx_v[j, sl]
            d = plsc.load_gather(den_v, [idx])
            val_v[j, sl] = val_v[j, sl] / (d + 1e-16)
        return carry

    lax.fori_loop(0, NCH, normbody, 0)
    pltpu.sync_copy(val_v, out_hbm.at[wid])


EPT = NCH * CHUNK
NBUF = 7


@functools.partial(
    pl.kernel,
    out_type=jax.ShapeDtypeStruct((NW, EPT, HID), jnp.float32),
    mesh=_sc_mesh,
    scratch_types=[
        pltpu.VMEM((NCH, CHUNK), jnp.int32),
        [pltpu.VMEM((CHUNK, HID), jnp.float32) for _ in range(NBUF)],
        pltpu.SemaphoreType.DMA,
        pltpu.SemaphoreType.DMA,
    ],
    compiler_params=pltpu.CompilerParams(use_tc_tiling_on_sc=False),
)
def _sc_gather(table_hbm, idx_hbm, out_hbm, idx_v, bufs, gsem, ssem):
    c = lax.axis_index("c")
    s = lax.axis_index("s")
    wid = s * NC + c
    pltpu.sync_copy(idx_hbm.at[wid], idx_v)

    def body(g, carry):
        for b in range(NBUF):
            j = g * NBUF + b
            pltpu.async_copy(table_hbm.at[idx_v.at[j]], bufs[b], gsem)
        for b in range(NBUF):
            j = g * NBUF + b
            pltpu.make_async_copy(table_hbm.at[idx_v.at[j]], bufs[b], gsem).wait()
            pltpu.async_copy(bufs[b], out_hbm.at[wid, pl.ds(j * CHUNK, CHUNK)], ssem)
        for b in range(NBUF):
            j = g * NBUF + b
            pltpu.make_async_copy(bufs[b], out_hbm.at[wid, pl.ds(j * CHUNK, CHUNK)], ssem).wait()
        return carry

    lax.fori_loop(0, NCH // NBUF, body, 0)


def _gather_rows_sc(table, idx_3d):
    return _sc_gather(table, idx_3d).reshape(E_PAD, HID)


HHID = HID // 4
NBUF_S = 14
FL_SL = SEG_SL // 2


@functools.partial(
    pl.kernel,
    out_type=jax.ShapeDtypeStruct((NC, 2, NSEG, HHID), jnp.float32),
    mesh=_sc_mesh,
    scratch_types=[
        pltpu.VMEM((NCH, CHUNK), jnp.int32),
        [pltpu.VMEM((CHUNK, HHID), jnp.float32) for _ in range(NBUF_S)],
        pltpu.VMEM((FL_SL, HHID), jnp.float32),
        pltpu.VMEM_SHARED((NSEG, HHID), jnp.float32),
        pltpu.SemaphoreType.DMA,
        pltpu.SemaphoreType.DMA,
    ],
    compiler_params=pltpu.CompilerParams(use_tc_tiling_on_sc=False),
)
def _sc_segsum_rows(za_hbm, dst_hbm, zeros_hbm, out_hbm,
                    idx_v, bufs, zbuf, acc, rsem, asem):
    c = lax.axis_index("c")
    s = lax.axis_index("s")

    def make_slice_body(base, col0):
        def body(g, carry):
            for b in range(NBUF_S):
                j = g * NBUF_S + b
                src = za_hbm.at[pl.ds(base + j * CHUNK, CHUNK),
                                pl.ds(col0, HHID)]
                pltpu.async_copy(src, bufs[b], rsem)
            for b in range(NBUF_S):
                j = g * NBUF_S + b
                src = za_hbm.at[pl.ds(base + j * CHUNK, CHUNK),
                                pl.ds(col0, HHID)]
                pltpu.make_async_copy(src, bufs[b], rsem).wait()
                pltpu.async_copy(bufs[b], acc.at[idx_v.at[j]], asem, add=True)
            for b in range(NBUF_S):
                j = g * NBUF_S + b
                pltpu.make_async_copy(
                    bufs[b], acc.at[idx_v.at[j]], asem).wait()
            return carry

        return body

    for q in range(2):
        col0 = pl.multiple_of((c * 2 + q) * HHID, 8)
        pltpu.sync_copy(zeros_hbm.at[pl.ds(0, FL_SL)], zbuf)
        for half in range(2):
            pltpu.sync_copy(
                zbuf, acc.at[pl.ds(s * SEG_SL + half * FL_SL, FL_SL)])
        plsc.subcore_barrier()
        for wid in (s * NC + c, s * NC + (1 - c)):
            pltpu.sync_copy(dst_hbm.at[wid], idx_v)
            lax.fori_loop(0, NCH // NBUF_S, make_slice_body(wid * EPT, col0), 0)
        plsc.subcore_barrier()
        for half in range(2):
            row0 = s * SEG_SL + half * FL_SL
            pltpu.sync_copy(acc.at[pl.ds(row0, FL_SL)], zbuf)
            pltpu.sync_copy(zbuf, out_hbm.at[c, q, pl.ds(row0, FL_SL)])


def _segsum_rows_sc(za, dst_3d, zeros_half):
    res = _sc_segsum_rows(za, dst_3d, zeros_half)
    return jnp.concatenate(
        [res[0, 0], res[0, 1], res[1, 0], res[1, 1]], axis=1)[:N_NODES]


def _softmax_norm_sc(ex, dst_3d, zeros_seg):
    ex_3d = ex.reshape(NW, NCH, CHUNK)
    return _sc_softmax_norm(ex_3d, dst_3d, zeros_seg).reshape(E_PAD)


def _linear(x, W, b):
    return x @ W.T + b


def _layernorm(x, g, b, eps=1e-5):
    m = jnp.mean(x, axis=-1, keepdims=True)
    v = jnp.mean((x - m) ** 2, axis=-1, keepdims=True)
    return (x - m) / jnp.sqrt(v + eps) * g + b


def _leaky(x, s):
    return jnp.where(x >= 0, x, s * x)


def _graph_norm(x, w, b, ms, eps=1e-5):
    mean = jnp.mean(x, axis=0, keepdims=True)
    out = x - ms * mean
    var = jnp.mean(out ** 2, axis=0, keepdims=True)
    return w * out / jnp.sqrt(var + eps) + b


def _klat_body(gd_ref, z_ref, scale_ref, g_ref, b_ref, out_ref):
    gd = gd_ref[...]
    scale = scale_ref[0]
    k = jnp.exp(-gd / (scale + 1e-06))
    k = jnp.where(gd > D_MAX, 0.0, k)
    m = jnp.dot(k, z_ref[...], preferred_element_type=jnp.float32)
    mu = jnp.mean(m, axis=-1, keepdims=True)
    var = jnp.mean((m - mu) ** 2, axis=-1, keepdims=True)
    out_ref[...] = (m - mu) / jnp.sqrt(var + 1e-5) * g_ref[...] + b_ref[...]


def _k_latent(gd, z_sensors, scale, g, b):
    blk = 1000
    grid = (N_NODES // blk,)
    return pl.pallas_call(
        _klat_body,
        grid=grid,
        in_specs=[
            pl.BlockSpec((blk, N_SENSORS), lambda i: (i, 0)),
            pl.BlockSpec((N_SENSORS, HID), lambda i: (0, 0)),
            pl.BlockSpec(memory_space=pltpu.SMEM),
            pl.BlockSpec((1, HID), lambda i: (0, 0)),
            pl.BlockSpec((1, HID), lambda i: (0, 0)),
        ],
        out_specs=pl.BlockSpec((blk, HID), lambda i: (i, 0)),
        out_shape=jax.ShapeDtypeStruct((N_NODES, HID), jnp.float32),
    )(gd, z_sensors, scale.reshape(1), g.reshape(1, HID), b.reshape(1, HID))


def _gat_layer(x, edge_attr_pad, p, pre, src_3d, dst_3d,
               padmask, zeros_seg, zeros_half):
    h = _linear(x, p[pre + '_Wn'], p[pre + '_bn'])
    e = _linear(edge_attr_pad, p[pre + '_We'], p[pre + '_be'])
    z = _gather_rows_sc(h, src_3d) * jax.nn.sigmoid(e)
    alpha = jnp.sum(z * p[pre + '_attn'], axis=-1)
    alpha = _leaky(alpha, 0.1)
    ex = jnp.exp(alpha) * padmask
    a = _softmax_norm_sc(ex, dst_3d, zeros_seg)
    return _segsum_rows_sc(z * a[:, None], dst_3d, zeros_half)


def _sensor_encoder(input_seq, p):
    S, T, _ = input_seq.shape
    h = jnp.zeros((S, HID), dtype=input_seq.dtype)
    h_new = h
    for t in range(T):
        x_t = _linear(input_seq[:, t, 0:1], p['enc_in_W'], p['enc_in_b'])
        delta = input_seq[:, t, 1]
        dh = (-h + jnp.tanh(x_t + _linear(h, p['enc_Wh_W'], p['enc_Wh_b']))) / p['enc_tau']
        h_new = h + delta[:, None] * dh
    return jnp.tanh(_linear(h_new, p['enc_out_W'], p['enc_out_b']))


def _w_xyz_mlp(x, p):
    h = jnp.tanh(_layernorm(_linear(x, p['wxyz_W1'], p['wxyz_b1']), p['wxyz_g1'], p['wxyz_e1']))
    h = jnp.tanh(_linear(h, p['wxyz_W2'], p['wxyz_b2']))
    return _linear(h, p['wxyz_W3'], p['wxyz_b3'])


def _w_type_mlp(x, p):
    h = jnp.tanh(_layernorm(_linear(x, p['wtype_W1'], p['wtype_b1']), p['wtype_g1'], p['wtype_e1']))
    h = jnp.tanh(_linear(h, p['wtype_W2'], p['wtype_b2']))
    return _linear(h, p['wtype_W3'], p['wtype_b3'])


def kernel(x, x_branch, geodesic_dist, ini_GNN_temp, edge_index, edge_attr, params):
    p = params
    temp_seq = x_branch[:, 6:]
    T = temp_seq.shape[1]
    delta_t = jnp.asarray([2.0] * (T - 1) + [120.0], dtype=jnp.float32)
    delta_t = delta_t / jnp.max(delta_t)
    delta_t = jnp.broadcast_to(delta_t[None, :], temp_seq.shape)
    input_seq = jnp.stack([temp_seq, delta_t], axis=-1)
    z_sensors = _sensor_encoder(input_seq, p)
    x_latent = _k_latent(geodesic_dist, z_sensors, p['kernel_scale'],
                         p['ln_lat_g'], p['ln_lat_b'])
    x_keep = x[:, jnp.asarray([0, 1, 2, 4, 5, 6])]
    xyz = x_keep[:, 0:3]
    x_xyz_w = _w_xyz_mlp(xyz, p) * xyz
    x_type = x_keep[:, 3:6]
    x_type_w = _w_type_mlp(x_type, p) * x_type
    x_ini = ini_GNN_temp * p['weights']
    x_all = jnp.concatenate([x_xyz_w, x_type_w, x_ini, x_latent], axis=1)
    n = x.shape[0]
    n_pad = E_PAD - edge_index.shape[1]
    pad = jnp.zeros((n_pad,), jnp.int32)
    src_3d = jnp.concatenate([edge_index[0], pad]).reshape(NW, NCH, CHUNK)
    dst_3d = jnp.concatenate([edge_index[1], pad]).reshape(NW, NCH, CHUNK)
    edge_attr_pad = jnp.concatenate(
        [edge_attr, jnp.zeros((n_pad, 3), jnp.float32)])
    padmask = jnp.concatenate([jnp.ones((edge_index.shape[1],), jnp.float32),
                               jnp.zeros((n_pad,), jnp.float32)])
    zeros_seg = jnp.zeros((NSEG,), jnp.float32)
    zeros_half = jnp.zeros((FL_SL, HHID), jnp.float32)
    gl = functools.partial(_gat_layer, edge_attr_pad=edge_attr_pad, p=p,
                           src_3d=src_3d, dst_3d=dst_3d, padmask=padmask,
                           zeros_seg=zeros_seg, zeros_half=zeros_half)
    h1 = _leaky(_graph_norm(gl(x_all, pre='g1'), p['n1_w'], p['n1_b'], p['n1_ms']), 0.01)
    h2 = _leaky(_graph_norm(gl(h1, pre='g2'), p['n2_w'], p['n2_b'], p['n2_ms']), 0.01) + h1
    h3 = _leaky(_graph_norm(gl(h2, pre='g3'), p['n3_w'], p['n3_b'], p['n3_ms']), 0.01) + h2
    h4 = _leaky(_graph_norm(gl(h3, pre='g4'), p['n4_w'], p['n4_b'], p['n4_ms']), 0.01) + h3
    h5 = _leaky(_graph_norm(gl(h4, pre='g5'), p['n5_w'], p['n5_b'], p['n5_ms']), 0.01) + h4
    o = _leaky(_layernorm(_linear(h5, p['out_W1'], p['out_b1']), p['out_g1'], p['out_e1']), 0.01)
    o = jnp.tanh(_linear(o, p['out_W2'], p['out_b2']))
    return o[:, 0]
